# baseline (speedup 1.0000x reference)
import functools

import jax
import jax.numpy as jnp
from jax import lax
from jax.experimental import pallas as pl
from jax.experimental.pallas import tpu as pltpu
from jax.experimental.pallas import tpu_sc as plsc

NC, NS = 2, 16
NW = NC * NS
LANES = 16


def _sc_mesh():
    return plsc.VectorSubcoreMesh(core_axis_name="c", subcore_axis_name="s")


@functools.partial(jax.jit, static_argnums=(1, 2))
def _degree(edges4, n_edges, n_pad):
    e_per = n_edges // NW
    B = 125
    nblk = e_per // B
    rpt = n_pad // NS

    @functools.partial(
        pl.kernel,
        out_type=jax.ShapeDtypeStruct((NC, n_pad, LANES), jnp.float32),
        mesh=_sc_mesh(),
        scratch_types=[
            pltpu.VMEM((nblk, B), jnp.int32),
            pltpu.VMEM((B, LANES), jnp.float32),
            pltpu.VMEM((rpt, LANES), jnp.float32),
            pltpu.VMEM_SHARED((n_pad, LANES), jnp.float32),
            pltpu.SemaphoreType.DMA,
        ],
        compiler_params=pltpu.CompilerParams(use_tc_tiling_on_sc=False),
    )
    def deg_kernel(edge_hbm, out_hbm, dstv, onesb, zbuf, deg_sh, sem):
        cid = lax.axis_index("c")
        sid = lax.axis_index("s")
        wid = sid * NC + cid

        pltpu.sync_copy(edge_hbm.at[1, wid], dstv)

        def zrow(i, _):
            zbuf[i, :] = jnp.zeros((LANES,), jnp.float32)
            return ()
        lax.fori_loop(0, rpt, zrow, ())

        def orow(i, _):
            onesb[i, :] = jnp.ones((LANES,), jnp.float32)
            return ()
        lax.fori_loop(0, B, orow, ())

        pltpu.sync_copy(zbuf, deg_sh.at[pl.ds(sid * rpt, rpt)])
        plsc.subcore_barrier()

        W = 16

        def body(j, _):
            pltpu.async_copy(onesb, deg_sh.at[dstv.at[j]], sem, add=True)

            @pl.when(j >= W)
            def _():
                pltpu.make_async_copy(onesb, deg_sh.at[dstv.at[j - W]],
                                      sem).wait()
            return ()
        lax.fori_loop(0, nblk, body, ())

        def drain(j, _):
            pltpu.make_async_copy(onesb, deg_sh.at[dstv.at[j]], sem).wait()
            return ()
        lax.fori_loop(nblk - W, nblk, drain, ())
        plsc.subcore_barrier()

        pltpu.sync_copy(deg_sh.at[pl.ds(sid * rpt, rpt)],
                        out_hbm.at[cid, pl.ds(sid * rpt, rpt)])

    return deg_kernel(edges4)


@functools.partial(jax.jit, static_argnums=(2, 3, 4))
def _propagate(hp, edges4, n_pad, n_edges, feat):
    e_per = n_edges // NW
    B = 125
    nblk = e_per // B
    NB = 8
    rpt = n_pad // NS
    ZR = 64
    nz = rpt // ZR

    @functools.partial(
        pl.kernel,
        out_type=jax.ShapeDtypeStruct((NC, n_pad, feat), jnp.float32),
        mesh=_sc_mesh(),
        scratch_types=[
            pltpu.VMEM((nblk, B), jnp.int32),
            pltpu.VMEM((nblk, B), jnp.int32),
            [pltpu.VMEM((B, feat), jnp.float32) for _ in range(NB)],
            pltpu.VMEM((ZR, feat), jnp.float32),
            pltpu.VMEM_SHARED((n_pad, feat), jnp.float32),
            [pltpu.SemaphoreType.DMA for _ in range(NB)],
            [pltpu.SemaphoreType.DMA for _ in range(NB)],
        ],
        compiler_params=pltpu.CompilerParams(use_tc_tiling_on_sc=False),
    )
    def prop_kernel(hp_hbm, edge_hbm, out_hbm,
                    srcv, dstv, rows, zbuf, acc_sh, semg, sems):
        cid = lax.axis_index("c")
        sid = lax.axis_index("s")
        wid = sid * NC + cid

        pltpu.sync_copy(edge_hbm.at[0, wid], srcv)
        pltpu.sync_copy(edge_hbm.at[1, wid], dstv)

        def wait_gather(j, s):
            pltpu.make_async_copy(hp_hbm.at[srcv.at[j]], rows[s], semg[s]).wait()

        def wait_scatter(j, s):
            pltpu.make_async_copy(rows[s], acc_sh.at[dstv.at[j]], sems[s]).wait()

        for s in range(NB):
            pltpu.async_copy(hp_hbm.at[srcv.at[s]], rows[s], semg[s])

        def zrow(i, _):
            for k in range(feat // LANES):
                zbuf[i, pl.ds(k * LANES, LANES)] = jnp.zeros((LANES,), jnp.float32)
            return ()
        lax.fori_loop(0, ZR, zrow, ())
        for k in range(nz):
            pltpu.sync_copy(zbuf, acc_sh.at[pl.ds(sid * rpt + k * ZR, ZR)])
        plsc.subcore_barrier()

        def body(jj, _):
            j0 = jj * NB
            for s in range(NB):
                wait_gather(j0 + s, s)
                pltpu.async_copy(rows[s], acc_sh.at[dstv.at[j0 + s]], sems[s],
                                 add=True)
            for s in range(NB):
                wait_scatter(j0 + s, s)
                pltpu.async_copy(hp_hbm.at[srcv.at[j0 + NB + s]], rows[s],
                                 semg[s])
            return ()
        lax.fori_loop(0, nblk // NB - 1, body, ())
        j0 = nblk - NB
        for s in range(NB):
            wait_gather(j0 + s, s)
            pltpu.async_copy(rows[s], acc_sh.at[dstv.at[j0 + s]], sems[s],
                             add=True)
        for s in range(NB):
            wait_scatter(j0 + s, s)
        plsc.subcore_barrier()

        pltpu.sync_copy(acc_sh.at[pl.ds(sid * rpt, rpt)],
                        out_hbm.at[cid, pl.ds(sid * rpt, rpt)])

    return prop_kernel(hp, edges4)


def _first_layer(x, w, deg_parts, blk):
    n, dft = x.shape
    h = w.shape[1]

    def body(x_ref, w_ref, deg_ref, o_ref, dv_ref):
        d = jnp.sum(deg_ref[...], axis=(0, 2)) * (1.0 / LANES)
        dv = lax.rsqrt(d + 1.0)[:, None]
        dv_ref[...] = dv
        o_ref[...] = dv * jnp.dot(
            x_ref[...], w_ref[...], preferred_element_type=jnp.float32)

    return pl.pallas_call(
        body,
        grid=(n // blk,),
        in_specs=[
            pl.BlockSpec((blk, dft), lambda i: (i, 0)),
            pl.BlockSpec((dft, h), lambda i: (0, 0)),
            pl.BlockSpec((NC, blk, LANES), lambda i: (0, i, 0)),
        ],
        out_specs=[
            pl.BlockSpec((blk, h), lambda i: (i, 0)),
            pl.BlockSpec((blk, 1), lambda i: (i, 0)),
        ],
        out_shape=[
            jax.ShapeDtypeStruct((n, h), jnp.float32),
            jax.ShapeDtypeStruct((n, 1), jnp.float32),
        ],
    )(x, w, deg_parts)


def _mid_layer(a0, a1, hp, dinv_col, b_row, w, blk):
    n, h = hp.shape
    h2 = w.shape[1]

    def body(a0_ref, a1_ref, hp_ref, dv_ref, b_ref, w_ref, o_ref):
        dv = dv_ref[...]
        z = dv * (a0_ref[...] + a1_ref[...] + hp_ref[...]) + b_ref[...]
        z = jnp.maximum(z, 0.0)
        o_ref[...] = dv * jnp.dot(z, w_ref[...],
                                  preferred_element_type=jnp.float32)

    return pl.pallas_call(
        body,
        grid=(n // blk,),
        in_specs=[
            pl.BlockSpec((blk, h), lambda i: (i, 0)),
            pl.BlockSpec((blk, h), lambda i: (i, 0)),
            pl.BlockSpec((blk, h), lambda i: (i, 0)),
            pl.BlockSpec((blk, 1), lambda i: (i, 0)),
            pl.BlockSpec((1, h), lambda i: (0, 0)),
            pl.BlockSpec((h, h2), lambda i: (0, 0)),
        ],
        out_specs=pl.BlockSpec((blk, h2), lambda i: (i, 0)),
        out_shape=jax.ShapeDtypeStruct((n, h2), jnp.float32),
    )(a0, a1, hp, dinv_col, b_row, w)


def _final_layer(a0, a1, hp, dinv_col, b_row, n_classes, blk):
    n, h = hp.shape

    def body(a0_ref, a1_ref, hp_ref, dv_ref, b_ref, o_ref):
        t = dv_ref[...] * (a0_ref[...] + a1_ref[...] + hp_ref[...])
        t = t[:, :n_classes] + b_ref[...]
        m = jnp.max(t, axis=1, keepdims=True)
        e = jnp.exp(t - m)
        lse = jnp.log(jnp.sum(e, axis=1, keepdims=True))
        o_ref[...] = t - m - lse

    return pl.pallas_call(
        body,
        grid=(n // blk,),
        in_specs=[
            pl.BlockSpec((blk, h), lambda i: (i, 0)),
            pl.BlockSpec((blk, h), lambda i: (i, 0)),
            pl.BlockSpec((blk, h), lambda i: (i, 0)),
            pl.BlockSpec((blk, 1), lambda i: (i, 0)),
            pl.BlockSpec((1, n_classes), lambda i: (0, 0)),
        ],
        out_specs=pl.BlockSpec((blk, n_classes), lambda i: (i, 0)),
        out_shape=jax.ShapeDtypeStruct((n, n_classes), jnp.float32),
    )(a0, a1, hp, dinv_col, b_row)


def kernel(x, edge_index, W1, b1, W2, b2, Wf, bf):
    n, _ = x.shape
    e = edge_index.shape[1]
    h = W1.shape[1]
    c = Wf.shape[1]
    blk = 400

    edges4 = edge_index.astype(jnp.int32).reshape(2, NW, 80, 125)

    n_pad = 10240
    deg_parts = _degree(edges4, e, n_pad)
    hp1, dinv_col = _first_layer(x, W1, deg_parts, blk)
    acc1 = _propagate(hp1, edges4, n_pad, e, h)
    hp2 = _mid_layer(acc1[0], acc1[1], hp1, dinv_col,
                     b1.reshape(1, h), W2, blk)
    acc2 = _propagate(hp2, edges4, n_pad, e, h)
    c_pad = 48
    wf_pad = jnp.pad(Wf, ((0, 0), (0, c_pad - c)))
    hp3 = _mid_layer(acc2[0], acc2[1], hp2, dinv_col,
                     b2.reshape(1, h), wf_pad, blk)
    acc3 = _propagate(hp3, edges4, n_pad, e, c_pad)
    return _final_layer(acc3[0], acc3[1], hp3, dinv_col,
                        bf.reshape(1, c), c, blk)


# speedup vs baseline: 1.1007x; 1.1007x over previous
import functools

import jax
import jax.numpy as jnp
from jax import lax
from jax.experimental import pallas as pl
from jax.experimental.pallas import tpu as pltpu
from jax.experimental.pallas import tpu_sc as plsc

NC, NS = 2, 16
NW = NC * NS
LANES = 16


def _sc_mesh():
    return plsc.VectorSubcoreMesh(core_axis_name="c", subcore_axis_name="s")


@functools.partial(jax.jit, static_argnums=(1, 2))
def _degree(edges4, n_edges, n_pad):
    e_per = n_edges // NW
    B = 125
    nblk = e_per // B
    rpt = n_pad // NS

    @functools.partial(
        pl.kernel,
        out_type=jax.ShapeDtypeStruct((NC, n_pad, LANES), jnp.float32),
        mesh=_sc_mesh(),
        scratch_types=[
            pltpu.VMEM((nblk, B), jnp.int32),
            pltpu.VMEM((B, LANES), jnp.float32),
            pltpu.VMEM((rpt, LANES), jnp.float32),
            pltpu.VMEM_SHARED((n_pad, LANES), jnp.float32),
            pltpu.SemaphoreType.DMA,
        ],
        compiler_params=pltpu.CompilerParams(use_tc_tiling_on_sc=False),
    )
    def deg_kernel(edge_hbm, out_hbm, dstv, onesb, zbuf, deg_sh, sem):
        cid = lax.axis_index("c")
        sid = lax.axis_index("s")
        wid = sid * NC + cid

        pltpu.sync_copy(edge_hbm.at[1, wid], dstv)

        def zrow(i, _):
            zbuf[i, :] = jnp.zeros((LANES,), jnp.float32)
            return ()
        lax.fori_loop(0, rpt, zrow, ())

        def orow(i, _):
            onesb[i, :] = jnp.ones((LANES,), jnp.float32)
            return ()
        lax.fori_loop(0, B, orow, ())

        pltpu.sync_copy(zbuf, deg_sh.at[pl.ds(sid * rpt, rpt)])
        plsc.subcore_barrier()

        W = 16

        def body(j, _):
            pltpu.async_copy(onesb, deg_sh.at[dstv.at[j]], sem, add=True)

            @pl.when(j >= W)
            def _():
                pltpu.make_async_copy(onesb, deg_sh.at[dstv.at[j - W]],
                                      sem).wait()
            return ()
        lax.fori_loop(0, nblk, body, ())

        def drain(j, _):
            pltpu.make_async_copy(onesb, deg_sh.at[dstv.at[j]], sem).wait()
            return ()
        lax.fori_loop(nblk - W, nblk, drain, ())
        plsc.subcore_barrier()

        pltpu.sync_copy(deg_sh.at[pl.ds(sid * rpt, rpt)],
                        out_hbm.at[cid, pl.ds(sid * rpt, rpt)])

    return deg_kernel(edges4)


@functools.partial(jax.jit, static_argnums=(2, 3, 4))
def _propagate(hp, edges4, n_pad, n_edges, feat):
    e_per = n_edges // NW
    B = 125
    nblk = e_per // B
    NB = 8
    rpt = n_pad // NS
    ZR = 64
    nz = rpt // ZR

    @functools.partial(
        pl.kernel,
        out_type=jax.ShapeDtypeStruct((NC, n_pad, feat), jnp.float32),
        mesh=_sc_mesh(),
        scratch_types=[
            pltpu.VMEM((nblk, B), jnp.int32),
            pltpu.VMEM((nblk, B), jnp.int32),
            [pltpu.VMEM((B, feat), jnp.float32) for _ in range(NB)],
            pltpu.VMEM((ZR, feat), jnp.float32),
            pltpu.VMEM_SHARED((n_pad, feat), jnp.float32),
            [pltpu.SemaphoreType.DMA for _ in range(NB)],
            [pltpu.SemaphoreType.DMA for _ in range(NB)],
        ],
        compiler_params=pltpu.CompilerParams(use_tc_tiling_on_sc=False),
    )
    def prop_kernel(hp_hbm, edge_hbm, out_hbm,
                    srcv, dstv, rows, zbuf, acc_sh, semg, sems):
        cid = lax.axis_index("c")
        sid = lax.axis_index("s")
        wid = sid * NC + cid

        pltpu.sync_copy(edge_hbm.at[0, wid], srcv)
        pltpu.sync_copy(edge_hbm.at[1, wid], dstv)

        def wait_gather(j, s):
            pltpu.make_async_copy(hp_hbm.at[srcv.at[j]], rows[s], semg[s]).wait()

        def wait_scatter(j, s):
            pltpu.make_async_copy(rows[s], acc_sh.at[dstv.at[j]], sems[s]).wait()

        for s in range(NB):
            pltpu.async_copy(hp_hbm.at[srcv.at[s]], rows[s], semg[s])

        def zrow(i, _):
            for k in range(feat // LANES):
                zbuf[i, pl.ds(k * LANES, LANES)] = jnp.zeros((LANES,), jnp.float32)
            return ()
        lax.fori_loop(0, ZR, zrow, ())
        for k in range(nz):
            pltpu.sync_copy(zbuf, acc_sh.at[pl.ds(sid * rpt + k * ZR, ZR)])
        plsc.subcore_barrier()

        def body(jj, _):
            j0 = jj * NB
            for s in range(NB):
                wait_gather(j0 + s, s)
                pltpu.async_copy(rows[s], acc_sh.at[dstv.at[j0 + s]], sems[s],
                                 add=True)
            for s in range(NB):
                wait_scatter(j0 + s, s)
                pltpu.async_copy(hp_hbm.at[srcv.at[j0 + NB + s]], rows[s],
                                 semg[s])
            return ()
        lax.fori_loop(0, nblk // NB - 1, body, ())
        j0 = nblk - NB
        for s in range(NB):
            wait_gather(j0 + s, s)
            pltpu.async_copy(rows[s], acc_sh.at[dstv.at[j0 + s]], sems[s],
                             add=True)
        for s in range(NB):
            wait_scatter(j0 + s, s)
        plsc.subcore_barrier()

        pltpu.sync_copy(acc_sh.at[pl.ds(sid * rpt, rpt)],
                        out_hbm.at[cid, pl.ds(sid * rpt, rpt)])

    return prop_kernel(hp, edges4)


def _first_layer(x, w, deg_parts, blk):
    n, dft = x.shape
    h = w.shape[1]

    def body(x_ref, w_ref, deg_ref, o_ref, dv_ref):
        d = jnp.sum(deg_ref[...], axis=(0, 2)) * (1.0 / LANES)
        dv = lax.rsqrt(d + 1.0)[:, None]
        dv_ref[...] = dv
        o_ref[...] = dv * jnp.dot(
            x_ref[...], w_ref[...], preferred_element_type=jnp.float32)

    return pl.pallas_call(
        body,
        grid=(n // blk,),
        in_specs=[
            pl.BlockSpec((blk, dft), lambda i: (i, 0)),
            pl.BlockSpec((dft, h), lambda i: (0, 0)),
            pl.BlockSpec((NC, blk, LANES), lambda i: (0, i, 0)),
        ],
        out_specs=[
            pl.BlockSpec((blk, h), lambda i: (i, 0)),
            pl.BlockSpec((blk, 1), lambda i: (i, 0)),
        ],
        out_shape=[
            jax.ShapeDtypeStruct((n, h), jnp.float32),
            jax.ShapeDtypeStruct((n, 1), jnp.float32),
        ],
    )(x, w, deg_parts)


def _mid_layer(a0, a1, hp, dinv_col, b_row, w, blk):
    n, h = hp.shape
    h2 = w.shape[1]

    def body(a0_ref, a1_ref, hp_ref, dv_ref, b_ref, w_ref, o_ref):
        dv = dv_ref[...]
        z = dv * (a0_ref[...] + a1_ref[...] + hp_ref[...]) + b_ref[...]
        z = jnp.maximum(z, 0.0)
        o_ref[...] = dv * jnp.dot(z, w_ref[...],
                                  preferred_element_type=jnp.float32)

    return pl.pallas_call(
        body,
        grid=(n // blk,),
        in_specs=[
            pl.BlockSpec((blk, h), lambda i: (i, 0)),
            pl.BlockSpec((blk, h), lambda i: (i, 0)),
            pl.BlockSpec((blk, h), lambda i: (i, 0)),
            pl.BlockSpec((blk, 1), lambda i: (i, 0)),
            pl.BlockSpec((1, h), lambda i: (0, 0)),
            pl.BlockSpec((h, h2), lambda i: (0, 0)),
        ],
        out_specs=pl.BlockSpec((blk, h2), lambda i: (i, 0)),
        out_shape=jax.ShapeDtypeStruct((n, h2), jnp.float32),
    )(a0, a1, hp, dinv_col, b_row, w)


def _final_layer(a0, a1, hp, dinv_col, b_row, n_classes, blk):
    n, h = hp.shape

    def body(a0_ref, a1_ref, hp_ref, dv_ref, b_ref, o_ref):
        t = dv_ref[...] * (a0_ref[...] + a1_ref[...] + hp_ref[...])
        t = t[:, :n_classes] + b_ref[...]
        m = jnp.max(t, axis=1, keepdims=True)
        e = jnp.exp(t - m)
        lse = jnp.log(jnp.sum(e, axis=1, keepdims=True))
        o_ref[...] = t - m - lse

    return pl.pallas_call(
        body,
        grid=(n // blk,),
        in_specs=[
            pl.BlockSpec((blk, h), lambda i: (i, 0)),
            pl.BlockSpec((blk, h), lambda i: (i, 0)),
            pl.BlockSpec((blk, h), lambda i: (i, 0)),
            pl.BlockSpec((blk, 1), lambda i: (i, 0)),
            pl.BlockSpec((1, n_classes), lambda i: (0, 0)),
        ],
        out_specs=pl.BlockSpec((blk, n_classes), lambda i: (i, 0)),
        out_shape=jax.ShapeDtypeStruct((n, n_classes), jnp.float32),
    )(a0, a1, hp, dinv_col, b_row)


def kernel(x, edge_index, W1, b1, W2, b2, Wf, bf):
    n, _ = x.shape
    e = edge_index.shape[1]
    h = W1.shape[1]
    c = Wf.shape[1]
    blk = 1000

    edges4 = edge_index.astype(jnp.int32).reshape(2, NW, 80, 125)

    n_pad = 10240
    deg_parts = _degree(edges4, e, n_pad)
    hp1, dinv_col = _first_layer(x, W1, deg_parts, blk)
    acc1 = _propagate(hp1, edges4, n_pad, e, h)
    hp2 = _mid_layer(acc1[0], acc1[1], hp1, dinv_col,
                     b1.reshape(1, h), W2, blk)
    acc2 = _propagate(hp2, edges4, n_pad, e, h)
    c_pad = 48
    wf_pad = jnp.pad(Wf, ((0, 0), (0, c_pad - c)))
    hp3 = _mid_layer(acc2[0], acc2[1], hp2, dinv_col,
                     b2.reshape(1, h), wf_pad, blk)
    acc3 = _propagate(hp3, edges4, n_pad, e, c_pad)
    return _final_layer(acc3[0], acc3[1], hp3, dinv_col,
                        bf.reshape(1, c), c, blk)


# speedup vs baseline: 1.1810x; 1.0729x over previous
import functools

import jax
import jax.numpy as jnp
from jax import lax
from jax.experimental import pallas as pl
from jax.experimental.pallas import tpu as pltpu
from jax.experimental.pallas import tpu_sc as plsc

NC, NS = 2, 16
NW = NC * NS
LANES = 16


def _sc_mesh():
    return plsc.VectorSubcoreMesh(core_axis_name="c", subcore_axis_name="s")


@functools.partial(jax.jit, static_argnums=(1, 2))
def _degree(edges4, n_edges, n_pad):
    e_per = n_edges // NW
    B = 125
    nblk = e_per // B
    rpt = n_pad // NS

    @functools.partial(
        pl.kernel,
        out_type=jax.ShapeDtypeStruct((NC, n_pad, LANES), jnp.float32),
        mesh=_sc_mesh(),
        scratch_types=[
            pltpu.VMEM((nblk, B), jnp.int32),
            pltpu.VMEM((B, LANES), jnp.float32),
            pltpu.VMEM((rpt, LANES), jnp.float32),
            pltpu.VMEM_SHARED((n_pad, LANES), jnp.float32),
            pltpu.SemaphoreType.DMA,
        ],
        compiler_params=pltpu.CompilerParams(use_tc_tiling_on_sc=False),
    )
    def deg_kernel(edge_hbm, out_hbm, dstv, onesb, zbuf, deg_sh, sem):
        cid = lax.axis_index("c")
        sid = lax.axis_index("s")
        wid = sid * NC + cid

        pltpu.sync_copy(edge_hbm.at[1, wid], dstv)

        def zrow(i, _):
            zbuf[i, :] = jnp.zeros((LANES,), jnp.float32)
            return ()
        lax.fori_loop(0, rpt, zrow, ())

        def orow(i, _):
            onesb[i, :] = jnp.ones((LANES,), jnp.float32)
            return ()
        lax.fori_loop(0, B, orow, ())

        pltpu.sync_copy(zbuf, deg_sh.at[pl.ds(sid * rpt, rpt)])
        plsc.subcore_barrier()

        W = 16

        def body(j, _):
            pltpu.async_copy(onesb, deg_sh.at[dstv.at[j]], sem, add=True)

            @pl.when(j >= W)
            def _():
                pltpu.make_async_copy(onesb, deg_sh.at[dstv.at[j - W]],
                                      sem).wait()
            return ()
        lax.fori_loop(0, nblk, body, ())

        def drain(j, _):
            pltpu.make_async_copy(onesb, deg_sh.at[dstv.at[j]], sem).wait()
            return ()
        lax.fori_loop(nblk - W, nblk, drain, ())
        plsc.subcore_barrier()

        pltpu.sync_copy(deg_sh.at[pl.ds(sid * rpt, rpt)],
                        out_hbm.at[cid, pl.ds(sid * rpt, rpt)])

    return deg_kernel(edges4)


@functools.partial(jax.jit, static_argnums=(2, 3, 4))
def _propagate(hp, edges4, n_pad, n_edges, feat):
    e_per = n_edges // NW
    B = 125
    nblk = e_per // B
    NB = 8
    rpt = n_pad // NS
    ZR = 64
    nz = rpt // ZR

    @functools.partial(
        pl.kernel,
        out_type=jax.ShapeDtypeStruct((NC, n_pad, feat), jnp.float32),
        mesh=_sc_mesh(),
        scratch_types=[
            pltpu.VMEM((nblk, B), jnp.int32),
            pltpu.VMEM((nblk, B), jnp.int32),
            [pltpu.VMEM((B, feat), jnp.float32) for _ in range(NB)],
            pltpu.VMEM((ZR, feat), jnp.float32),
            pltpu.VMEM_SHARED((n_pad, feat), jnp.float32),
            [pltpu.SemaphoreType.DMA for _ in range(NB)],
            [pltpu.SemaphoreType.DMA for _ in range(NB)],
        ],
        compiler_params=pltpu.CompilerParams(use_tc_tiling_on_sc=False),
    )
    def prop_kernel(hp_hbm, edge_hbm, out_hbm,
                    srcv, dstv, rows, zbuf, acc_sh, semg, sems):
        cid = lax.axis_index("c")
        sid = lax.axis_index("s")
        wid = sid * NC + cid

        pltpu.sync_copy(edge_hbm.at[0, wid], srcv)
        pltpu.sync_copy(edge_hbm.at[1, wid], dstv)

        def wait_gather(j, s):
            pltpu.make_async_copy(hp_hbm.at[srcv.at[j]], rows[s], semg[s]).wait()

        def wait_scatter(j, s):
            pltpu.make_async_copy(rows[s], acc_sh.at[dstv.at[j]], sems[s]).wait()

        for s in range(NB):
            pltpu.async_copy(hp_hbm.at[srcv.at[s]], rows[s], semg[s])

        def zrow(i, _):
            for k in range(feat // LANES):
                zbuf[i, pl.ds(k * LANES, LANES)] = jnp.zeros((LANES,), jnp.float32)
            return ()
        lax.fori_loop(0, ZR, zrow, ())
        for k in range(nz):
            pltpu.sync_copy(zbuf, acc_sh.at[pl.ds(sid * rpt + k * ZR, ZR)])
        plsc.subcore_barrier()

        def body(jj, _):
            j0 = jj * NB
            for s in range(NB):
                wait_gather(j0 + s, s)
                pltpu.async_copy(rows[s], acc_sh.at[dstv.at[j0 + s]], sems[s],
                                 add=True)
            for s in range(NB):
                wait_scatter(j0 + s, s)
                pltpu.async_copy(hp_hbm.at[srcv.at[j0 + NB + s]], rows[s],
                                 semg[s])
            return ()
        lax.fori_loop(0, nblk // NB - 1, body, ())
        j0 = nblk - NB
        for s in range(NB):
            wait_gather(j0 + s, s)
            pltpu.async_copy(rows[s], acc_sh.at[dstv.at[j0 + s]], sems[s],
                             add=True)
        for s in range(NB):
            wait_scatter(j0 + s, s)
        plsc.subcore_barrier()

        pltpu.sync_copy(acc_sh.at[pl.ds(sid * rpt, rpt)],
                        out_hbm.at[cid, pl.ds(sid * rpt, rpt)])

    return prop_kernel(hp, edges4)


def _first_layer(x, w, deg_parts, blk):
    n, dft = x.shape
    h = w.shape[1]

    def body(x_ref, w_ref, deg_ref, o_ref, dv_ref):
        d = jnp.sum(deg_ref[...], axis=(0, 2)) * (1.0 / LANES)
        dv = lax.rsqrt(d + 1.0)[:, None]
        dv_ref[...] = dv
        o_ref[...] = dv * jnp.dot(
            x_ref[...], w_ref[...], preferred_element_type=jnp.float32)

    return pl.pallas_call(
        body,
        grid=(n // blk,),
        in_specs=[
            pl.BlockSpec((blk, dft), lambda i: (i, 0)),
            pl.BlockSpec((dft, h), lambda i: (0, 0)),
            pl.BlockSpec((NC, blk, LANES), lambda i: (0, i, 0)),
        ],
        out_specs=[
            pl.BlockSpec((blk, h), lambda i: (i, 0)),
            pl.BlockSpec((blk, 1), lambda i: (i, 0)),
        ],
        out_shape=[
            jax.ShapeDtypeStruct((n, h), jnp.float32),
            jax.ShapeDtypeStruct((n, 1), jnp.float32),
        ],
    )(x, w, deg_parts)


def _mid_layer(acc, hp, dinv_col, b_row, w, blk):
    n, h = hp.shape
    h2 = w.shape[1]

    def body(a_ref, hp_ref, dv_ref, b_ref, w_ref, o_ref):
        dv = dv_ref[...]
        a = a_ref[...]
        z = dv * (a[0] + a[1] + hp_ref[...]) + b_ref[...]
        z = jnp.maximum(z, 0.0)
        o_ref[...] = dv * jnp.dot(z, w_ref[...],
                                  preferred_element_type=jnp.float32)

    return pl.pallas_call(
        body,
        grid=(n // blk,),
        in_specs=[
            pl.BlockSpec((NC, blk, h), lambda i: (0, i, 0)),
            pl.BlockSpec((blk, h), lambda i: (i, 0)),
            pl.BlockSpec((blk, 1), lambda i: (i, 0)),
            pl.BlockSpec((1, h), lambda i: (0, 0)),
            pl.BlockSpec((h, h2), lambda i: (0, 0)),
        ],
        out_specs=pl.BlockSpec((blk, h2), lambda i: (i, 0)),
        out_shape=jax.ShapeDtypeStruct((n, h2), jnp.float32),
    )(acc, hp, dinv_col, b_row, w)


def _final_layer(acc, hp, dinv_col, b_row, n_classes, blk):
    n, h = hp.shape

    def body(a_ref, hp_ref, dv_ref, b_ref, o_ref):
        a = a_ref[...]
        t = dv_ref[...] * (a[0] + a[1] + hp_ref[...])
        t = t[:, :n_classes] + b_ref[...]
        m = jnp.max(t, axis=1, keepdims=True)
        e = jnp.exp(t - m)
        lse = jnp.log(jnp.sum(e, axis=1, keepdims=True))
        o_ref[...] = t - m - lse

    return pl.pallas_call(
        body,
        grid=(n // blk,),
        in_specs=[
            pl.BlockSpec((NC, blk, h), lambda i: (0, i, 0)),
            pl.BlockSpec((blk, h), lambda i: (i, 0)),
            pl.BlockSpec((blk, 1), lambda i: (i, 0)),
            pl.BlockSpec((1, n_classes), lambda i: (0, 0)),
        ],
        out_specs=pl.BlockSpec((blk, n_classes), lambda i: (i, 0)),
        out_shape=jax.ShapeDtypeStruct((n, n_classes), jnp.float32),
    )(acc, hp, dinv_col, b_row)


def kernel(x, edge_index, W1, b1, W2, b2, Wf, bf):
    n, _ = x.shape
    e = edge_index.shape[1]
    h = W1.shape[1]
    c = Wf.shape[1]
    blk = 1000

    edges4 = edge_index.astype(jnp.int32).reshape(2, NW, 80, 125)

    n_pad = 10240
    deg_parts = _degree(edges4, e, n_pad)
    hp1, dinv_col = _first_layer(x, W1, deg_parts, blk)
    acc1 = _propagate(hp1, edges4, n_pad, e, h)
    hp2 = _mid_layer(acc1, hp1, dinv_col, b1.reshape(1, h), W2, blk)
    acc2 = _propagate(hp2, edges4, n_pad, e, h)
    c_pad = 48
    wf_pad = jnp.pad(Wf, ((0, 0), (0, c_pad - c)))
    hp3 = _mid_layer(acc2, hp2, dinv_col,
                     b2.reshape(1, h), wf_pad, blk)
    acc3 = _propagate(hp3, edges4, n_pad, e, c_pad)
    return _final_layer(acc3, hp3, dinv_col, bf.reshape(1, c), c, blk)


# speedup vs baseline: 1.2192x; 1.0324x over previous
import functools

import jax
import jax.numpy as jnp
from jax import lax
from jax.experimental import pallas as pl
from jax.experimental.pallas import tpu as pltpu
from jax.experimental.pallas import tpu_sc as plsc

NC, NS = 2, 16
NW = NC * NS
LANES = 16


def _sc_mesh():
    return plsc.VectorSubcoreMesh(core_axis_name="c", subcore_axis_name="s")


@functools.partial(jax.jit, static_argnums=(1, 2))
def _degree(edges4, n_edges, n_pad):
    e_per = n_edges // NW
    B = 125
    nblk = e_per // B
    rpt = n_pad // NS

    @functools.partial(
        pl.kernel,
        out_type=jax.ShapeDtypeStruct((NC, n_pad, LANES), jnp.float32),
        mesh=_sc_mesh(),
        scratch_types=[
            pltpu.VMEM((nblk, B), jnp.int32),
            pltpu.VMEM((B, LANES), jnp.float32),
            pltpu.VMEM((rpt, LANES), jnp.float32),
            pltpu.VMEM_SHARED((n_pad, LANES), jnp.float32),
            pltpu.SemaphoreType.DMA,
        ],
        compiler_params=pltpu.CompilerParams(use_tc_tiling_on_sc=False),
    )
    def deg_kernel(edge_hbm, out_hbm, dstv, onesb, zbuf, deg_sh, sem):
        cid = lax.axis_index("c")
        sid = lax.axis_index("s")
        wid = sid * NC + cid

        pltpu.sync_copy(edge_hbm.at[1, wid], dstv)

        def zrow(i, _):
            zbuf[i, :] = jnp.zeros((LANES,), jnp.float32)
            return ()
        lax.fori_loop(0, rpt, zrow, ())

        def orow(i, _):
            onesb[i, :] = jnp.ones((LANES,), jnp.float32)
            return ()
        lax.fori_loop(0, B, orow, ())

        pltpu.sync_copy(zbuf, deg_sh.at[pl.ds(sid * rpt, rpt)])
        plsc.subcore_barrier()

        W = 16

        def body(j, _):
            pltpu.async_copy(onesb, deg_sh.at[dstv.at[j]], sem, add=True)

            @pl.when(j >= W)
            def _():
                pltpu.make_async_copy(onesb, deg_sh.at[dstv.at[j - W]],
                                      sem).wait()
            return ()
        lax.fori_loop(0, nblk, body, ())

        def drain(j, _):
            pltpu.make_async_copy(onesb, deg_sh.at[dstv.at[j]], sem).wait()
            return ()
        lax.fori_loop(nblk - W, nblk, drain, ())
        plsc.subcore_barrier()

        pltpu.sync_copy(deg_sh.at[pl.ds(sid * rpt, rpt)],
                        out_hbm.at[cid, pl.ds(sid * rpt, rpt)])

    return deg_kernel(edges4)


@functools.partial(jax.jit, static_argnums=(2, 3, 4))
def _propagate(hp, edges4, n_pad, n_edges, feat):
    e_per = n_edges // NW
    B = 125
    nblk = e_per // B
    NB = 8
    rpt = n_pad // NS
    ZR = 64
    nz = rpt // ZR

    @functools.partial(
        pl.kernel,
        out_type=jax.ShapeDtypeStruct((NC, n_pad, feat), jnp.float32),
        mesh=_sc_mesh(),
        scratch_types=[
            pltpu.VMEM((nblk, B), jnp.int32),
            pltpu.VMEM((nblk, B), jnp.int32),
            [pltpu.VMEM((B, feat), jnp.float32) for _ in range(NB)],
            pltpu.VMEM((ZR, feat), jnp.float32),
            pltpu.VMEM_SHARED((n_pad, feat), jnp.float32),
            [pltpu.SemaphoreType.DMA for _ in range(NB)],
            [pltpu.SemaphoreType.DMA for _ in range(NB)],
        ],
        compiler_params=pltpu.CompilerParams(use_tc_tiling_on_sc=False),
    )
    def prop_kernel(hp_hbm, edge_hbm, out_hbm,
                    srcv, dstv, rows, zbuf, acc_sh, semg, sems):
        cid = lax.axis_index("c")
        sid = lax.axis_index("s")
        wid = sid * NC + cid

        pltpu.sync_copy(edge_hbm.at[0, wid], srcv)
        pltpu.sync_copy(edge_hbm.at[1, wid], dstv)

        def wait_gather(j, s):
            pltpu.make_async_copy(hp_hbm.at[srcv.at[j]], rows[s], semg[s]).wait()

        def wait_scatter(j, s):
            pltpu.make_async_copy(rows[s], acc_sh.at[dstv.at[j]], sems[s]).wait()

        for s in range(NB):
            pltpu.async_copy(hp_hbm.at[srcv.at[s]], rows[s], semg[s])

        def zrow(i, _):
            for k in range(feat // LANES):
                zbuf[i, pl.ds(k * LANES, LANES)] = jnp.zeros((LANES,), jnp.float32)
            return ()
        lax.fori_loop(0, ZR, zrow, ())
        for k in range(nz):
            pltpu.sync_copy(zbuf, acc_sh.at[pl.ds(sid * rpt + k * ZR, ZR)])
        plsc.subcore_barrier()

        def body(jj, _):
            j0 = jj * NB
            for s in range(NB):
                wait_gather(j0 + s, s)
                pltpu.async_copy(rows[s], acc_sh.at[dstv.at[j0 + s]], sems[s],
                                 add=True)
            for s in range(NB):
                wait_scatter(j0 + s, s)
                pltpu.async_copy(hp_hbm.at[srcv.at[j0 + NB + s]], rows[s],
                                 semg[s])
            return ()
        lax.fori_loop(0, nblk // NB - 1, body, ())
        j0 = nblk - NB
        for s in range(NB):
            wait_gather(j0 + s, s)
            pltpu.async_copy(rows[s], acc_sh.at[dstv.at[j0 + s]], sems[s],
                             add=True)
        for s in range(NB):
            wait_scatter(j0 + s, s)
        plsc.subcore_barrier()

        pltpu.sync_copy(acc_sh.at[pl.ds(sid * rpt, rpt)],
                        out_hbm.at[cid, pl.ds(sid * rpt, rpt)])

    return prop_kernel(hp, edges4)


def _first_layer(x, w, deg_parts, blk):
    n, dft = x.shape
    h = w.shape[1]

    def body(x_ref, w_ref, deg_ref, o_ref, dv_ref):
        d = jnp.sum(deg_ref[...], axis=(0, 2)) * (1.0 / LANES)
        dv = lax.rsqrt(d + 1.0)[:, None]
        dv_ref[...] = dv
        o_ref[...] = dv * jnp.dot(
            x_ref[...], w_ref[...], preferred_element_type=jnp.float32)

    return pl.pallas_call(
        body,
        grid=(n // blk,),
        in_specs=[
            pl.BlockSpec((blk, dft), lambda i: (i, 0)),
            pl.BlockSpec((dft, h), lambda i: (0, 0)),
            pl.BlockSpec((NC, blk, LANES), lambda i: (0, i, 0)),
        ],
        out_specs=[
            pl.BlockSpec((blk, h), lambda i: (i, 0)),
            pl.BlockSpec((blk, 1), lambda i: (i, 0)),
        ],
        out_shape=[
            jax.ShapeDtypeStruct((n, h), jnp.float32),
            jax.ShapeDtypeStruct((n, 1), jnp.float32),
        ],
    )(x, w, deg_parts)


def _mid_layer(acc, hp, dinv_col, b_row, w, blk):
    n, h = hp.shape
    h2 = w.shape[1]

    def body(a_ref, hp_ref, dv_ref, b_ref, w_ref, o_ref):
        dv = dv_ref[...]
        a = a_ref[...]
        z = dv * (a[0] + a[1] + hp_ref[...]) + b_ref[...]
        z = jnp.maximum(z, 0.0)
        o_ref[...] = dv * jnp.dot(z, w_ref[...],
                                  preferred_element_type=jnp.float32)

    return pl.pallas_call(
        body,
        grid=(n // blk,),
        in_specs=[
            pl.BlockSpec((NC, blk, h), lambda i: (0, i, 0)),
            pl.BlockSpec((blk, h), lambda i: (i, 0)),
            pl.BlockSpec((blk, 1), lambda i: (i, 0)),
            pl.BlockSpec((1, h), lambda i: (0, 0)),
            pl.BlockSpec((h, h2), lambda i: (0, 0)),
        ],
        out_specs=pl.BlockSpec((blk, h2), lambda i: (i, 0)),
        out_shape=jax.ShapeDtypeStruct((n, h2), jnp.float32),
    )(acc, hp, dinv_col, b_row, w)


def _final_layer(acc, hp, dinv_col, b_row, n_classes, blk):
    n, h = hp.shape

    def body(a_ref, hp_ref, dv_ref, b_ref, o_ref):
        a = a_ref[...]
        t = dv_ref[...] * (a[0] + a[1] + hp_ref[...])
        t = t[:, :n_classes] + b_ref[...]
        m = jnp.max(t, axis=1, keepdims=True)
        e = jnp.exp(t - m)
        lse = jnp.log(jnp.sum(e, axis=1, keepdims=True))
        o_ref[...] = t - m - lse

    return pl.pallas_call(
        body,
        grid=(n // blk,),
        in_specs=[
            pl.BlockSpec((NC, blk, h), lambda i: (0, i, 0)),
            pl.BlockSpec((blk, h), lambda i: (i, 0)),
            pl.BlockSpec((blk, 1), lambda i: (i, 0)),
            pl.BlockSpec((1, n_classes), lambda i: (0, 0)),
        ],
        out_specs=pl.BlockSpec((blk, n_classes), lambda i: (i, 0)),
        out_shape=jax.ShapeDtypeStruct((n, n_classes), jnp.float32),
    )(acc, hp, dinv_col, b_row)


def kernel(x, edge_index, W1, b1, W2, b2, Wf, bf):
    n, _ = x.shape
    e = edge_index.shape[1]
    h = W1.shape[1]
    c = Wf.shape[1]
    blk = 2000

    edges4 = edge_index.astype(jnp.int32).reshape(2, NW, 80, 125)

    n_pad = 10240
    deg_parts = _degree(edges4, e, n_pad)
    hp1, dinv_col = _first_layer(x, W1, deg_parts, blk)
    acc1 = _propagate(hp1, edges4, n_pad, e, h)
    hp2 = _mid_layer(acc1, hp1, dinv_col, b1.reshape(1, h), W2, blk)
    acc2 = _propagate(hp2, edges4, n_pad, e, h)
    c_pad = 48
    wf_pad = jnp.pad(Wf, ((0, 0), (0, c_pad - c)))
    hp3 = _mid_layer(acc2, hp2, dinv_col,
                     b2.reshape(1, h), wf_pad, blk)
    acc3 = _propagate(hp3, edges4, n_pad, e, c_pad)
    return _final_layer(acc3, hp3, dinv_col, bf.reshape(1, c), c, blk)


# speedup vs baseline: 1.2357x; 1.0135x over previous
import functools

import jax
import jax.numpy as jnp
from jax import lax
from jax.experimental import pallas as pl
from jax.experimental.pallas import tpu as pltpu
from jax.experimental.pallas import tpu_sc as plsc

NC, NS = 2, 16
NW = NC * NS
LANES = 16


def _sc_mesh():
    return plsc.VectorSubcoreMesh(core_axis_name="c", subcore_axis_name="s")


@functools.partial(jax.jit, static_argnums=(1, 2))
def _degree(edges4, n_edges, n_pad):
    e_per = n_edges // NW
    B = 125
    nblk = e_per // B
    rpt = n_pad // NS

    @functools.partial(
        pl.kernel,
        out_type=jax.ShapeDtypeStruct((NC, n_pad, LANES), jnp.float32),
        mesh=_sc_mesh(),
        scratch_types=[
            pltpu.VMEM((nblk, B), jnp.int32),
            pltpu.VMEM((B, LANES), jnp.float32),
            pltpu.VMEM((rpt, LANES), jnp.float32),
            pltpu.VMEM_SHARED((n_pad, LANES), jnp.float32),
            pltpu.SemaphoreType.DMA,
        ],
        compiler_params=pltpu.CompilerParams(use_tc_tiling_on_sc=False),
    )
    def deg_kernel(edge_hbm, out_hbm, dstv, onesb, zbuf, deg_sh, sem):
        cid = lax.axis_index("c")
        sid = lax.axis_index("s")
        wid = sid * NC + cid

        pltpu.sync_copy(edge_hbm.at[1, wid], dstv)

        def zrow(i, _):
            zbuf[i, :] = jnp.zeros((LANES,), jnp.float32)
            return ()
        lax.fori_loop(0, rpt, zrow, ())

        def orow(i, _):
            onesb[i, :] = jnp.ones((LANES,), jnp.float32)
            return ()
        lax.fori_loop(0, B, orow, ())

        pltpu.sync_copy(zbuf, deg_sh.at[pl.ds(sid * rpt, rpt)])
        plsc.subcore_barrier()

        W = 16

        def body(j, _):
            pltpu.async_copy(onesb, deg_sh.at[dstv.at[j]], sem, add=True)

            @pl.when(j >= W)
            def _():
                pltpu.make_async_copy(onesb, deg_sh.at[dstv.at[j - W]],
                                      sem).wait()
            return ()
        lax.fori_loop(0, nblk, body, ())

        def drain(j, _):
            pltpu.make_async_copy(onesb, deg_sh.at[dstv.at[j]], sem).wait()
            return ()
        lax.fori_loop(nblk - W, nblk, drain, ())
        plsc.subcore_barrier()

        pltpu.sync_copy(deg_sh.at[pl.ds(sid * rpt, rpt)],
                        out_hbm.at[cid, pl.ds(sid * rpt, rpt)])

    return deg_kernel(edges4)


@functools.partial(jax.jit, static_argnums=(2, 3, 4))
def _propagate(hp, edges4, n_pad, n_edges, feat):
    e_per = n_edges // NW
    B = 125
    nblk = e_per // B
    NB = 8
    rpt = n_pad // NS
    ZR = 64
    nz = rpt // ZR

    @functools.partial(
        pl.kernel,
        out_type=jax.ShapeDtypeStruct((NC, n_pad, feat), jnp.float32),
        mesh=_sc_mesh(),
        scratch_types=[
            pltpu.VMEM((nblk, B), jnp.int32),
            pltpu.VMEM((nblk, B), jnp.int32),
            [pltpu.VMEM((B, feat), jnp.float32) for _ in range(NB)],
            pltpu.VMEM((ZR, feat), jnp.float32),
            pltpu.VMEM_SHARED((n_pad, feat), jnp.float32),
            [pltpu.SemaphoreType.DMA for _ in range(NB)],
            [pltpu.SemaphoreType.DMA for _ in range(NB)],
        ],
        compiler_params=pltpu.CompilerParams(use_tc_tiling_on_sc=False),
    )
    def prop_kernel(hp_hbm, edge_hbm, out_hbm,
                    srcv, dstv, rows, zbuf, acc_sh, semg, sems):
        cid = lax.axis_index("c")
        sid = lax.axis_index("s")
        wid = sid * NC + cid

        pltpu.sync_copy(edge_hbm.at[0, wid], srcv)
        pltpu.sync_copy(edge_hbm.at[1, wid], dstv)

        def wait_gather(j, s):
            pltpu.make_async_copy(hp_hbm.at[srcv.at[j]], rows[s], semg[s]).wait()

        def wait_scatter(j, s):
            pltpu.make_async_copy(rows[s], acc_sh.at[dstv.at[j]], sems[s]).wait()

        for s in range(NB):
            pltpu.async_copy(hp_hbm.at[srcv.at[s]], rows[s], semg[s])

        def zrow(i, _):
            for k in range(feat // LANES):
                zbuf[i, pl.ds(k * LANES, LANES)] = jnp.zeros((LANES,), jnp.float32)
            return ()
        lax.fori_loop(0, ZR, zrow, ())
        for k in range(nz):
            pltpu.sync_copy(zbuf, acc_sh.at[pl.ds(sid * rpt + k * ZR, ZR)])
        plsc.subcore_barrier()

        def body(jj, _):
            j0 = jj * NB
            for s in range(NB):
                wait_gather(j0 + s, s)
                pltpu.async_copy(rows[s], acc_sh.at[dstv.at[j0 + s]], sems[s],
                                 add=True)
            for s in range(NB):
                wait_scatter(j0 + s, s)
                pltpu.async_copy(hp_hbm.at[srcv.at[j0 + NB + s]], rows[s],
                                 semg[s])
            return ()
        lax.fori_loop(0, nblk // NB - 1, body, ())
        j0 = nblk - NB
        for s in range(NB):
            wait_gather(j0 + s, s)
            pltpu.async_copy(rows[s], acc_sh.at[dstv.at[j0 + s]], sems[s],
                             add=True)
        for s in range(NB):
            wait_scatter(j0 + s, s)
        plsc.subcore_barrier()

        pltpu.sync_copy(acc_sh.at[pl.ds(sid * rpt, rpt)],
                        out_hbm.at[cid, pl.ds(sid * rpt, rpt)])

    return prop_kernel(hp, edges4)


def _first_layer(x, w, deg_parts, blk):
    n, dft = x.shape
    h = w.shape[1]

    def body(x_ref, w_ref, deg_ref, o_ref, dv_ref):
        d = jnp.sum(deg_ref[...], axis=(0, 2)) * (1.0 / LANES)
        dv = lax.rsqrt(d + 1.0)[:, None]
        dv_ref[...] = dv
        o_ref[...] = dv * jnp.dot(
            x_ref[...], w_ref[...], preferred_element_type=jnp.float32)

    return pl.pallas_call(
        body,
        grid=(n // blk,),
        in_specs=[
            pl.BlockSpec((blk, dft), lambda i: (i, 0)),
            pl.BlockSpec((dft, h), lambda i: (0, 0)),
            pl.BlockSpec((NC, blk, LANES), lambda i: (0, i, 0)),
        ],
        out_specs=[
            pl.BlockSpec((blk, h), lambda i: (i, 0)),
            pl.BlockSpec((blk, 1), lambda i: (i, 0)),
        ],
        out_shape=[
            jax.ShapeDtypeStruct((n, h), jnp.float32),
            jax.ShapeDtypeStruct((n, 1), jnp.float32),
        ],
    )(x, w, deg_parts)


def _mid_layer(acc, hp, dinv_col, b_row, w, blk):
    n, h = hp.shape
    h2 = w.shape[1]

    def body(a_ref, hp_ref, dv_ref, b_ref, w_ref, o_ref):
        dv = dv_ref[...]
        a = a_ref[...]
        z = dv * (a[0] + a[1] + hp_ref[...]) + b_ref[...]
        z = jnp.maximum(z, 0.0)
        o_ref[...] = dv * jnp.dot(z, w_ref[...],
                                  preferred_element_type=jnp.float32)

    return pl.pallas_call(
        body,
        grid=(n // blk,),
        in_specs=[
            pl.BlockSpec((NC, blk, h), lambda i: (0, i, 0)),
            pl.BlockSpec((blk, h), lambda i: (i, 0)),
            pl.BlockSpec((blk, 1), lambda i: (i, 0)),
            pl.BlockSpec((1, h), lambda i: (0, 0)),
            pl.BlockSpec((h, h2), lambda i: (0, 0)),
        ],
        out_specs=pl.BlockSpec((blk, h2), lambda i: (i, 0)),
        out_shape=jax.ShapeDtypeStruct((n, h2), jnp.float32),
    )(acc, hp, dinv_col, b_row, w)


def _final_layer(acc, hp, dinv_col, b_row, n_classes, blk):
    n, h = hp.shape

    def body(a_ref, hp_ref, dv_ref, b_ref, o_ref):
        a = a_ref[...]
        t = dv_ref[...] * (a[0] + a[1] + hp_ref[...])
        t = t[:, :n_classes] + b_ref[...]
        m = jnp.max(t, axis=1, keepdims=True)
        e = jnp.exp(t - m)
        lse = jnp.log(jnp.sum(e, axis=1, keepdims=True))
        o_ref[...] = t - m - lse

    return pl.pallas_call(
        body,
        grid=(n // blk,),
        in_specs=[
            pl.BlockSpec((NC, blk, h), lambda i: (0, i, 0)),
            pl.BlockSpec((blk, h), lambda i: (i, 0)),
            pl.BlockSpec((blk, 1), lambda i: (i, 0)),
            pl.BlockSpec((1, n_classes), lambda i: (0, 0)),
        ],
        out_specs=pl.BlockSpec((blk, n_classes), lambda i: (i, 0)),
        out_shape=jax.ShapeDtypeStruct((n, n_classes), jnp.float32),
    )(acc, hp, dinv_col, b_row)


def kernel(x, edge_index, W1, b1, W2, b2, Wf, bf):
    n, _ = x.shape
    e = edge_index.shape[1]
    h = W1.shape[1]
    c = Wf.shape[1]
    blk = 5000

    edges4 = edge_index.astype(jnp.int32).reshape(2, NW, 80, 125)

    n_pad = 10240
    deg_parts = _degree(edges4, e, n_pad)
    hp1, dinv_col = _first_layer(x, W1, deg_parts, blk)
    acc1 = _propagate(hp1, edges4, n_pad, e, h)
    hp2 = _mid_layer(acc1, hp1, dinv_col, b1.reshape(1, h), W2, blk)
    acc2 = _propagate(hp2, edges4, n_pad, e, h)
    c_pad = 48
    wf_pad = jnp.pad(Wf, ((0, 0), (0, c_pad - c)))
    hp3 = _mid_layer(acc2, hp2, dinv_col,
                     b2.reshape(1, h), wf_pad, blk)
    acc3 = _propagate(hp3, edges4, n_pad, e, c_pad)
    return _final_layer(acc3, hp3, dinv_col, bf.reshape(1, c), c, blk)
